# Initial kernel scaffold; baseline (speedup 1.0000x reference)
import jax
import jax.numpy as jnp
from jax.experimental import pallas as pl


def kernel(inputs, fixed_wv_weight, trainable_wv_weight):
    raise NotImplementedError("write your pallas kernel here")


# speedup vs baseline: 4.5050x; 4.5050x over previous
import functools

import jax
import jax.numpy as jnp
from jax import lax
from jax.experimental import pallas as pl
from jax.experimental.pallas import tpu as pltpu
from jax.experimental.pallas import tpu_sc as plsc

_NUM_FIXED = 998976
_NUM_TRAINABLE = 1024
_WV_DIM = 32

_NUM_WORKERS = 32
_CHUNK = 1024
_SUBGATHER = 128
_LANES = 16


def _emb_body(idx_hbm, fixed_hbm, train_hbm, out_hbm,
              idx_raw, idx_dma, rows, train_v, acc_v, sem_g):
    num_cores = 2
    wid = lax.axis_index("s") * num_cores + lax.axis_index("c")
    per_worker = idx_hbm.shape[0] // _NUM_WORKERS
    num_steps = per_worker // _CHUNK
    worker_base = wid * per_worker

    ones = jnp.full((_LANES,), 1, jnp.int32)
    zeros = jnp.full((_LANES,), 0, jnp.int32)

    pltpu.sync_copy(train_hbm, train_v)

    def step_body(g, carry):
        gbase = worker_base + g * _CHUNK
        pltpu.sync_copy(idx_hbm.at[pl.ds(gbase, _CHUNK)], idx_raw)

        def clamp_body(j, c0):
            v = idx_raw[pl.ds(j * _LANES, _LANES)]
            m = v >= _NUM_FIXED
            idx_dma[pl.ds(j * _LANES, _LANES)] = jnp.where(m, _NUM_FIXED - 1, v)
            acc_v[...] = acc_v[...] + jnp.where(m, ones, zeros)
            return c0

        acc_v[...] = zeros
        lax.fori_loop(0, _CHUNK // _LANES, clamp_body, 0)
        step_hits = jnp.sum(acc_v[...])

        descs = []
        for i in range(_CHUNK // _SUBGATHER):
            descs.append(pltpu.async_copy(
                fixed_hbm.at[idx_dma.at[pl.ds(i * _SUBGATHER, _SUBGATHER)]],
                rows.at[pl.ds(i * _SUBGATHER, _SUBGATHER)],
                sem_g))
        for d in descs:
            d.wait()

        @pl.when(step_hits > 0)
        def _patch():
            def patch_body(j, c1):
                p = j * _LANES
                v = idx_raw[pl.ds(p, _LANES)]
                m = v >= _NUM_FIXED
                cnt = jnp.sum(jnp.where(m, ones, zeros))

                @pl.when(cnt > 0)
                def _():
                    t = jnp.maximum(v - _NUM_FIXED, 0)
                    r = p + lax.iota(jnp.int32, _LANES)
                    for c in range(_WV_DIM):
                        cvec = jnp.full((_LANES,), c, jnp.int32)
                        vals = plsc.load_gather(train_v, [t, cvec], mask=m)
                        plsc.store_scatter(rows, [r, cvec], vals, mask=m)

                return c1

            lax.fori_loop(0, _CHUNK // _LANES, patch_body, 0)

        pltpu.sync_copy(rows, out_hbm.at[pl.ds(gbase, _CHUNK)])
        return carry

    lax.fori_loop(0, num_steps, step_body, 0)


def kernel(inputs, fixed_wv_weight, trainable_wv_weight):
    batch, hist = inputs.shape
    n_total = batch * hist
    idx_flat = inputs.reshape(n_total)

    mesh = plsc.VectorSubcoreMesh(core_axis_name="c", subcore_axis_name="s")
    run = functools.partial(
        pl.kernel,
        out_type=jax.ShapeDtypeStruct((n_total, _WV_DIM), jnp.float32),
        mesh=mesh,
        compiler_params=pltpu.CompilerParams(use_tc_tiling_on_sc=False,
                                             needs_layout_passes=False),
        scratch_types=[
            pltpu.VMEM((_CHUNK,), jnp.int32),
            pltpu.VMEM((_CHUNK,), jnp.int32),
            pltpu.VMEM((_CHUNK, _WV_DIM), jnp.float32),
            pltpu.VMEM((_NUM_TRAINABLE, _WV_DIM), jnp.float32),
            pltpu.VMEM((_LANES,), jnp.int32),
            pltpu.SemaphoreType.DMA,
        ],
    )(_emb_body)
    out = run(idx_flat, fixed_wv_weight, trainable_wv_weight)
    return out.reshape(batch, hist, _WV_DIM)


# speedup vs baseline: 5.0217x; 1.1147x over previous
import functools

import jax
import jax.numpy as jnp
from jax import lax
from jax.experimental import pallas as pl
from jax.experimental.pallas import tpu as pltpu
from jax.experimental.pallas import tpu_sc as plsc

_NUM_FIXED = 998976
_NUM_TRAINABLE = 1024
_WV_DIM = 32

_NUM_WORKERS = 32
_CHUNK = 512
_SUBGATHER = 128
_LANES = 16
_NBUF = 4


def _emb_body(idx_hbm, fixed_hbm, train_hbm, out_hbm,
              idx_raw, idx_dma, rows, train_v, acc_v, *sems):
    sem_i = sems[0:_NBUF]
    sem_g = sems[_NBUF:2 * _NBUF]
    sem_o = sems[2 * _NBUF:3 * _NBUF]

    num_cores = 2
    wid = lax.axis_index("s") * num_cores + lax.axis_index("c")
    per_worker = idx_hbm.shape[0] // _NUM_WORKERS
    num_steps = per_worker // _CHUNK
    worker_base = wid * per_worker

    ones = jnp.full((_LANES,), 1, jnp.int32)
    zeros = jnp.full((_LANES,), 0, jnp.int32)

    pltpu.sync_copy(train_hbm, train_v)

    def fire_idx(g, b):
        gbase = worker_base + g * _CHUNK
        pltpu.async_copy(idx_hbm.at[pl.ds(gbase, _CHUNK)], idx_raw.at[b],
                         sem_i[b])

    def wait_idx(b):
        pltpu.make_async_copy(idx_hbm.at[pl.ds(0, _CHUNK)], idx_raw.at[b],
                              sem_i[b]).wait()

    def clamp(b):
        def clamp_body(j, c0):
            v = idx_raw[b, pl.ds(j * _LANES, _LANES)]
            m = v >= _NUM_FIXED
            idx_dma[b, pl.ds(j * _LANES, _LANES)] = (
                jnp.where(m, _NUM_FIXED - 1, v))
            acc_v[b, :] = acc_v[b, :] + jnp.where(m, ones, zeros)
            return c0

        acc_v[b, :] = zeros
        lax.fori_loop(0, _CHUNK // _LANES, clamp_body, 0)

    def fire_gather(b):
        for i in range(_CHUNK // _SUBGATHER):
            pltpu.async_copy(
                fixed_hbm.at[idx_dma.at[b, pl.ds(i * _SUBGATHER, _SUBGATHER)]],
                rows.at[b, pl.ds(i * _SUBGATHER, _SUBGATHER)],
                sem_g[b])

    def wait_gather(b):
        pltpu.make_async_copy(fixed_hbm.at[pl.ds(0, _CHUNK)], rows.at[b],
                              sem_g[b]).wait()

    def fire_out(g, b):
        gbase = worker_base + g * _CHUNK
        pltpu.async_copy(rows.at[b], out_hbm.at[pl.ds(gbase, _CHUNK)],
                         sem_o[b])

    def wait_out(b):
        pltpu.make_async_copy(rows.at[b], out_hbm.at[pl.ds(0, _CHUNK)],
                              sem_o[b]).wait()

    def patch(b):
        step_hits = jnp.sum(acc_v[b, :])

        @pl.when(step_hits > 0)
        def _patch():
            def patch_body(j, c1):
                p = j * _LANES
                v = idx_raw[b, pl.ds(p, _LANES)]
                m = v >= _NUM_FIXED
                cnt = jnp.sum(jnp.where(m, ones, zeros))

                @pl.when(cnt > 0)
                def _():
                    t = jnp.maximum(v - _NUM_FIXED, 0)
                    r = p + lax.iota(jnp.int32, _LANES)
                    for c in range(_WV_DIM):
                        cvec = jnp.full((_LANES,), c, jnp.int32)
                        vals = plsc.load_gather(train_v, [t, cvec], mask=m)
                        plsc.store_scatter(rows.at[b], [r, cvec], vals,
                                           mask=m)

                return c1

            lax.fori_loop(0, _CHUNK // _LANES, patch_body, 0)

    fire_idx(0, 0)
    fire_idx(1, 1)
    wait_idx(0)
    clamp(0)
    fire_gather(0)

    def group_body(gg, carry):
        for b0 in range(_NBUF):
            g = gg * _NBUF + b0
            b1 = (b0 + 1) % _NBUF
            b2 = (b0 + 2) % _NBUF

            @pl.when(g + 2 < num_steps)
            def _():
                fire_idx(g + 2, b2)

            @pl.when(g + 1 < num_steps)
            def _():
                wait_idx(b1)
                clamp(b1)

                @pl.when(g + 1 >= _NBUF)
                def _():
                    wait_out(b1)

                fire_gather(b1)

            wait_gather(b0)
            patch(b0)
            fire_out(g, b0)
        return carry

    lax.fori_loop(0, num_steps // _NBUF, group_body, 0)

    for b in range(_NBUF):
        wait_out(b)


def kernel(inputs, fixed_wv_weight, trainable_wv_weight):
    batch, hist = inputs.shape
    n_total = batch * hist
    idx_flat = inputs.reshape(n_total)

    mesh = plsc.VectorSubcoreMesh(core_axis_name="c", subcore_axis_name="s")
    run = functools.partial(
        pl.kernel,
        out_type=jax.ShapeDtypeStruct((n_total, _WV_DIM), jnp.float32),
        mesh=mesh,
        compiler_params=pltpu.CompilerParams(use_tc_tiling_on_sc=False,
                                             needs_layout_passes=False),
        scratch_types=[
            pltpu.VMEM((_NBUF, _CHUNK), jnp.int32),
            pltpu.VMEM((_NBUF, _CHUNK), jnp.int32),
            pltpu.VMEM((_NBUF, _CHUNK, _WV_DIM), jnp.float32),
            pltpu.VMEM((_NUM_TRAINABLE, _WV_DIM), jnp.float32),
            pltpu.VMEM((_NBUF, _LANES), jnp.int32),
        ] + [pltpu.SemaphoreType.DMA] * (3 * _NBUF),
    )(_emb_body)
    out = run(idx_flat, fixed_wv_weight, trainable_wv_weight)
    return out.reshape(batch, hist, _WV_DIM)
